# Initial kernel scaffold; baseline (speedup 1.0000x reference)
import functools

import jax
import jax.numpy as jnp
from jax.experimental import pallas as pl
from jax.experimental.pallas import tpu as pltpu

N = 10000
E = 320000
D = 128
ED = 16
H = 4
C = 32
HC = H * C


def _matmul_kernel(x_ref, w_ref, o_ref):
    o_ref[...] = jnp.dot(x_ref[...], w_ref[...],
                         preferred_element_type=jnp.float32)


def _matmul(x, w):
    m, k = x.shape
    k2, n = w.shape
    bm = 1024
    grid = (pl.cdiv(m, bm),)
    return pl.pallas_call(
        _matmul_kernel,
        grid=grid,
        in_specs=[pl.BlockSpec((bm, k), lambda i: (i, 0)),
                  pl.BlockSpec((k, n), lambda i: (0, 0))],
        out_specs=pl.BlockSpec((bm, n), lambda i: (i, 0)),
        out_shape=jax.ShapeDtypeStruct((m, n), jnp.float32),
    )(x, w)


def kernel(x, edge_index, edge_attr, W0, att_src0, att_dst0, We0, att_e0, b0,
           W1, att_src1, att_dst1, We1, att_e1, b1, Wd1, bd1, Wd2, bd2):
    src = edge_index[0]
    dst = edge_index[1]

    S0 = (W0.reshape(D, H, C) * att_src0[None]).sum(-1)
    T0 = (W0.reshape(D, H, C) * att_dst0[None]).sum(-1)
    Ae0 = (We0.reshape(ED, H, C) * att_e0[None]).sum(-1)
    S1 = (W1.reshape(HC, H, C) * att_src1[None]).sum(-1)
    T1 = (W1.reshape(HC, H, C) * att_dst1[None]).sum(-1)
    Ae1 = (We1.reshape(ED, H, C) * att_e1[None]).sum(-1)

    aeE0 = edge_attr @ Ae0
    aeE1 = edge_attr @ Ae1

    ones = jnp.ones((E,), jnp.float32)
    deg = jax.ops.segment_sum(ones, dst, num_segments=N)
    segAe0 = jax.ops.segment_sum(aeE0, dst, num_segments=N)
    segAe1 = jax.ops.segment_sum(aeE1, dst, num_segments=N)
    invdeg = 1.0 / jnp.maximum(deg, 1.0)
    loop_ae0 = segAe0 * invdeg[:, None]
    loop_ae1 = segAe1 * invdeg[:, None]

    def layer(xin, W, S, T, aeE, loop_ae, b):
        h = _matmul(xin, W)
        a_s = xin @ S
        a_d = xin @ T
        bound = (jnp.max(a_s, axis=0) + jnp.max(a_d, axis=0)
                 + jnp.maximum(jnp.max(aeE, axis=0), 0.0))
        Bh = jnp.where(bound >= 0, bound, 0.2 * bound)

        alpha = a_s[src] + a_d[dst] + aeE
        alpha = jnp.where(alpha >= 0, alpha, 0.2 * alpha)
        p = jnp.exp(alpha - Bh[None])
        alpha_loop = a_s + a_d + loop_ae
        alpha_loop = jnp.where(alpha_loop >= 0, alpha_loop, 0.2 * alpha_loop)
        p_loop = jnp.exp(alpha_loop - Bh[None])

        denom = jax.ops.segment_sum(p, dst, num_segments=N) + p_loop
        msg = h[src].reshape(E, H, C) * p[:, :, None]
        acc = jax.ops.segment_sum(msg.reshape(E, HC), dst, num_segments=N)
        acc = acc + h * jnp.repeat(p_loop, C, axis=1)
        out = acc / jnp.repeat(denom, C, axis=1) + b[None]
        return jax.nn.relu(out)

    h1 = layer(x, W0, S0, T0, aeE0, loop_ae0, b0)
    h2 = layer(h1, W1, S1, T1, aeE1, loop_ae1, b1)

    g = h2.mean(axis=0, keepdims=True)
    z = jax.nn.relu(g @ Wd1 + bd1)
    return z @ Wd2 + bd2


# speedup vs baseline: 17.3491x; 17.3491x over previous
import functools

import jax
import jax.numpy as jnp
from jax import lax
from jax.experimental import pallas as pl
from jax.experimental.pallas import tpu as pltpu
from jax.experimental.pallas import tpu_sc as plsc

N = 10000
E = 320000
D = 128
ED = 16
H = 4
C = 32
HC = H * C

NC = 2
NS = 16
NW = NC * NS
K = 128
EPAD = 323584
EW = EPAD // NW
NCHUNK = EW // K
NPAD = 10112
RPW = NPAD // NS

_F32 = jnp.float32


def _node_body(x_ref, w_ref, asf_ref, adf_ref, hs0_ref, hs1_ref, aux_ref,
               mx_ref, s_ref):
    i = pl.program_id(0)
    h = jnp.dot(x_ref[...], w_ref[...], preferred_element_type=_F32)
    hs0_ref[...] = h[:, 0:HC // 2]
    hs1_ref[...] = h[:, HC // 2:HC]
    krow = lax.broadcasted_iota(jnp.int32, (HC, H), 0)
    jcol = lax.broadcasted_iota(jnp.int32, (HC, H), 1)
    rm = jnp.where(krow // C == jcol, 1.0, 0.0).astype(_F32)
    a_s = jnp.dot(h * asf_ref[...], rm, preferred_element_type=_F32)
    a_d = jnp.dot(h * adf_ref[...], rm, preferred_element_type=_F32)
    aux = jnp.concatenate([a_s, a_d], axis=1)
    aux_ref[...] = aux

    bm = jnp.max(aux, axis=0, keepdims=True)

    @pl.when(i == 0)
    def _():
        s_ref[0:1, 0:8] = bm

    @pl.when(i > 0)
    def _():
        s_ref[0:1, 0:8] = jnp.maximum(s_ref[0:1, 0:8], bm)

    @pl.when(i == pl.num_programs(0) - 1)
    def _():
        mx_ref[...] = s_ref[0:1, 0:8]


def _node_matmul(x, w, asf, adf):
    bm = 1000
    grid = (N // bm,)
    return pl.pallas_call(
        _node_body,
        grid=grid,
        in_specs=[pl.BlockSpec((bm, D), lambda i: (i, 0)),
                  pl.BlockSpec((D, HC), lambda i: (0, 0)),
                  pl.BlockSpec((1, HC), lambda i: (0, 0)),
                  pl.BlockSpec((1, HC), lambda i: (0, 0))],
        out_specs=[pl.BlockSpec((bm, HC // 2), lambda i: (i, 0)),
                   pl.BlockSpec((bm, HC // 2), lambda i: (i, 0)),
                   pl.BlockSpec((bm, 2 * H), lambda i: (i, 0)),
                   pl.BlockSpec((1, 2 * H), lambda i: (0, 0))],
        out_shape=[jax.ShapeDtypeStruct((N, HC // 2), _F32),
                   jax.ShapeDtypeStruct((N, HC // 2), _F32),
                   jax.ShapeDtypeStruct((N, 2 * H), _F32),
                   jax.ShapeDtypeStruct((1, 2 * H), _F32)],
        scratch_shapes=[pltpu.VMEM((8, 128), _F32)],
    )(x, w, asf, adf)


def _edge_body(ea_ref, we0_ref, ae0f_ref, we1_ref, ae1f_ref, o_ref, mx_ref,
               s_ref):
    i = pl.program_id(0)
    krow = lax.broadcasted_iota(jnp.int32, (HC, H), 0)
    jcol = lax.broadcasted_iota(jnp.int32, (HC, H), 1)
    rm = jnp.where(krow // C == jcol, 1.0, 0.0).astype(_F32)
    t0 = jnp.dot(we0_ref[...] * ae0f_ref[...], rm,
                 preferred_element_type=_F32)
    t1 = jnp.dot(we1_ref[...] * ae1f_ref[...], rm,
                 preferred_element_type=_F32)
    aep = jnp.concatenate([t0, t1, jnp.zeros((ED, 8), _F32)], axis=1)
    col = lax.broadcasted_iota(jnp.int32, (1, 16), 1)
    bias = jnp.where(col == 8, 1.0, 0.0).astype(_F32)
    out = jnp.dot(ea_ref[...], aep, preferred_element_type=_F32) + bias
    o_ref[...] = out

    bm = jnp.max(out, axis=0, keepdims=True)

    @pl.when(i == 0)
    def _():
        s_ref[0:1, 0:16] = bm

    @pl.when(i > 0)
    def _():
        s_ref[0:1, 0:16] = jnp.maximum(s_ref[0:1, 0:16], bm)

    @pl.when(i == pl.num_programs(0) - 1)
    def _():
        mx_ref[...] = s_ref[0:1, 0:16]


def _edge_logits(eap, We0, ae0f, We1, ae1f):
    be = 2048
    grid = (EPAD // be,)
    return pl.pallas_call(
        _edge_body,
        grid=grid,
        in_specs=[pl.BlockSpec((be, ED), lambda i: (i, 0)),
                  pl.BlockSpec((ED, HC), lambda i: (0, 0)),
                  pl.BlockSpec((1, HC), lambda i: (0, 0)),
                  pl.BlockSpec((ED, HC), lambda i: (0, 0)),
                  pl.BlockSpec((1, HC), lambda i: (0, 0))],
        out_specs=[pl.BlockSpec((be, 16), lambda i: (i, 0)),
                   pl.BlockSpec((1, 16), lambda i: (0, 0))],
        out_shape=[jax.ShapeDtypeStruct((EPAD, 16), _F32),
                   jax.ShapeDtypeStruct((1, 16), _F32)],
        scratch_shapes=[pltpu.VMEM((8, 128), _F32)],
    )(eap, We0, ae0f, We1, ae1f)


def _sc_stats_body(aeEp_hbm, dstp_hbm, z16_hbm, out_hbm, aeV, dstv, acc_sh):
    cid = lax.axis_index("c")
    sid = lax.axis_index("s")
    r0 = sid * RPW
    pltpu.sync_copy(z16_hbm, acc_sh.at[pl.ds(r0, RPW)])
    plsc.subcore_barrier()
    base = (cid * NS + sid) * EW

    def chunk(i, c):
        off = base + i * K
        pltpu.sync_copy(dstp_hbm.at[pl.ds(off, K)], dstv)
        pltpu.sync_copy(aeEp_hbm.at[pl.ds(off, K)], aeV)
        pltpu.sync_copy(aeV, acc_sh.at[dstv], add=True)
        return c

    lax.fori_loop(0, NCHUNK, chunk, 0)
    plsc.subcore_barrier()
    pltpu.sync_copy(acc_sh.at[pl.ds(r0, RPW)],
                    out_hbm.at[cid, pl.ds(r0, RPW)])


def _sc_stats(aeEp, dstp, z16):
    mesh = plsc.VectorSubcoreMesh(core_axis_name="c", subcore_axis_name="s")
    f = functools.partial(
        pl.kernel,
        out_type=jax.ShapeDtypeStruct((NC, NPAD, 16), _F32),
        mesh=mesh,
        compiler_params=pltpu.CompilerParams(use_tc_tiling_on_sc=False, needs_layout_passes=False),
        scratch_types=[
            pltpu.VMEM((K, 16), _F32),
            pltpu.VMEM((K,), jnp.int32),
            pltpu.VMEM_SHARED((NPAD, 16), _F32),
        ],
    )(_sc_stats_body)
    return f(aeEp, dstp, z16)


EWC = EPAD // NS
NCHUNKC = EWC // K
HH = HC // 2
NH0 = 5056
NPADH = 5120
RPWH = NPADH // NS


def _make_sc_layer(lcol, half):
    outr = NH0 if half == 0 else N - NH0

    def body(hs0_hbm, hs1_hbm, aux_hbm, srcp_hbm, dstp_hbm, aeEp_hbm,
             bnd_hbm, z64_hbm, z8_hbm, acc_out, den_out,
             table, rows, pbuf, aeV, srcv, dstv, dstl, bndv, acc_sh, den_sh):
        cid = lax.axis_index("c")
        sid = lax.axis_index("s")
        r0 = sid * RPWH
        pltpu.sync_copy(z64_hbm, acc_sh.at[pl.ds(r0, RPWH)])
        pltpu.sync_copy(z8_hbm, den_sh.at[pl.ds(r0, RPWH)])
        pltpu.sync_copy(aux_hbm, table)
        pltpu.sync_copy(bnd_hbm, bndv)

        iota16 = lax.iota(jnp.int32, 16)

        def zero_pbuf(z, c):
            rowv = z * 2 + iota16 // 8
            colv = iota16 % 8
            plsc.store_scatter(pbuf, [rowv, colv], jnp.zeros((16,), _F32))
            return c

        lax.fori_loop(0, K // 2, zero_pbuf, 0)
        plsc.subcore_barrier()

        base = sid * EWC
        bvec = bndv[:]
        hg0 = (cid * 2).astype(jnp.int32)

        def chunk(i, c):
            off = base + i * K
            pltpu.sync_copy(srcp_hbm.at[pl.ds(off, K)], srcv)
            pltpu.sync_copy(dstp_hbm.at[pl.ds(off, K)], dstv)
            pltpu.sync_copy(aeEp_hbm.at[pl.ds(off, K)], aeV)

            @pl.when(cid == 0)
            def _():
                pltpu.sync_copy(hs0_hbm.at[srcv], rows)

            @pl.when(cid == 1)
            def _():
                pltpu.sync_copy(hs1_hbm.at[srcv], rows)

            for g in range(K // 16):
                dv = dstv[pl.ds(g * 16, 16)]
                lv = dv - half * NH0
                ok = (lv >= 0) & (lv < NH0)
                dstl[pl.ds(g * 16, 16)] = jnp.where(ok, lv, NH0)

            for g in range(K // 16):
                sv = srcv[pl.ds(g * 16, 16)]
                dv = dstv[pl.ds(g * 16, 16)]
                for hl in range(2):
                    hg = hg0 + hl
                    hgv = jnp.full((16,), hg, jnp.int32)
                    a_s = plsc.load_gather(table, [sv, hgv])
                    a_d = plsc.load_gather(table, [dv, hgv + H])
                    ae = plsc.load_gather(
                        aeV, [g * 16 + iota16, hgv + 4 * lcol])
                    al = a_s + a_d + ae
                    al = jnp.where(al >= 0, al, 0.2 * al)
                    bsel = jnp.where(cid == 0, bvec[hl], bvec[2 + hl])
                    p = jnp.exp(al - bsel)
                    plsc.store_scatter(pbuf, [g * 16 + iota16, hgv], p)
                    for j in range(16):
                        e = g * 16 + j
                        w = p[j]
                        c0 = hl * C
                        rows[e, pl.ds(c0, 16)] = rows[e, pl.ds(c0, 16)] * w
                        rows[e, pl.ds(c0 + 16, 16)] = (
                            rows[e, pl.ds(c0 + 16, 16)] * w)

            pltpu.sync_copy(rows, acc_sh.at[dstl], add=True)
            pltpu.sync_copy(pbuf, den_sh.at[dstl], add=True)
            return c

        lax.fori_loop(0, NCHUNKC, chunk, 0)
        plsc.subcore_barrier()
        tail = outr - (NS - 1) * RPWH

        @pl.when(sid < NS - 1)
        def _():
            pltpu.sync_copy(acc_sh.at[pl.ds(r0, RPWH)],
                            acc_out.at[cid, pl.ds(r0, RPWH)])
            pltpu.sync_copy(den_sh.at[pl.ds(r0, RPWH)],
                            den_out.at[cid, pl.ds(r0, RPWH)])

        @pl.when(sid == NS - 1)
        def _():
            pltpu.sync_copy(acc_sh.at[pl.ds((NS - 1) * RPWH, tail)],
                            acc_out.at[cid, pl.ds((NS - 1) * RPWH, tail)])
            pltpu.sync_copy(den_sh.at[pl.ds((NS - 1) * RPWH, tail)],
                            den_out.at[cid, pl.ds((NS - 1) * RPWH, tail)])

    mesh = plsc.VectorSubcoreMesh(core_axis_name="c", subcore_axis_name="s")
    return functools.partial(
        pl.kernel,
        out_type=(jax.ShapeDtypeStruct((NC, outr, HH), _F32),
                  jax.ShapeDtypeStruct((NC, outr, 8), _F32)),
        mesh=mesh,
        compiler_params=pltpu.CompilerParams(use_tc_tiling_on_sc=False, needs_layout_passes=False),
        scratch_types=[
            pltpu.VMEM((N, 2 * H), _F32),
            pltpu.VMEM((K, HH), _F32),
            pltpu.VMEM((K, 8), _F32),
            pltpu.VMEM((K, 16), _F32),
            pltpu.VMEM((K,), jnp.int32),
            pltpu.VMEM((K,), jnp.int32),
            pltpu.VMEM((K,), jnp.int32),
            pltpu.VMEM((16,), _F32),
            pltpu.VMEM_SHARED((NPADH, HH), _F32),
            pltpu.VMEM_SHARED((NPADH, 8), _F32),
        ],
    )(body)


_sc_layers = [[_make_sc_layer(0, 0), _make_sc_layer(0, 1)],
              [_make_sc_layer(1, 0), _make_sc_layer(1, 1)]]


def _combine_body(lcol, acc_ref, den_ref, hs0_ref,
                  hs1_ref, aux_ref, seg0_ref, seg1_ref, bnd_ref, b_ref,
                  o_ref):
    seg = seg0_ref[...] + seg1_ref[...]
    deg = seg[:, 8:9]
    invd = 1.0 / jnp.maximum(deg, 1.0)
    lae = seg[:, 4 * lcol:4 * lcol + 4] * invd
    a = aux_ref[:, 0:4] + aux_ref[:, 4:8] + lae
    a = jnp.where(a >= 0, a, 0.2 * a)
    p = jnp.exp(a - bnd_ref[...])
    den = den_ref[:, 0:4] + p
    r = 1.0 / (den + 1e-16)
    jrow = lax.broadcasted_iota(jnp.int32, (H, HC), 0)
    kcol = lax.broadcasted_iota(jnp.int32, (H, HC), 1)
    erep = jnp.where(jrow == kcol // C, 1.0, 0.0).astype(_F32)
    racc = jnp.dot(r, erep, preferred_element_type=_F32)
    wrep = jnp.dot(p * r, erep, preferred_element_type=_F32)
    h = jnp.concatenate([hs0_ref[...], hs1_ref[...]], axis=1)
    out = acc_ref[...] * racc + h * wrep + b_ref[...]
    o_ref[...] = jnp.maximum(out, 0.0)


def _combine(lcol, acc, den, hs0, hs1, aux, seg0, seg1, bnd, b):
    bm = 1000
    grid = (N // bm,)
    return pl.pallas_call(
        functools.partial(_combine_body, lcol),
        grid=grid,
        in_specs=[pl.BlockSpec((bm, HC), lambda i: (i, 0)),
                  pl.BlockSpec((bm, 8), lambda i: (i, 0)),
                  pl.BlockSpec((bm, HH), lambda i: (i, 0)),
                  pl.BlockSpec((bm, HH), lambda i: (i, 0)),
                  pl.BlockSpec((bm, 2 * H), lambda i: (i, 0)),
                  pl.BlockSpec((bm, 16), lambda i: (i, 0)),
                  pl.BlockSpec((bm, 16), lambda i: (i, 0)),
                  pl.BlockSpec((1, H), lambda i: (0, 0)),
                  pl.BlockSpec((1, HC), lambda i: (0, 0))],
        out_specs=pl.BlockSpec((bm, HC), lambda i: (i, 0)),
        out_shape=jax.ShapeDtypeStruct((N, HC), _F32),
    )(acc, den, hs0, hs1, aux, seg0, seg1, bnd, b)


def _head_body(h_ref, wd1_ref, bd1_ref, wd2_ref, bd2_ref, o_ref, s_ref):
    i = pl.program_id(0)
    part = jnp.sum(h_ref[...], axis=0, keepdims=True)

    @pl.when(i == 0)
    def _():
        s_ref[0:1, :] = part

    @pl.when(i > 0)
    def _():
        s_ref[0:1, :] = s_ref[0:1, :] + part

    @pl.when(i == pl.num_programs(0) - 1)
    def _():
        g = s_ref[0:1, :] * (1.0 / N)
        z = jnp.dot(g, wd1_ref[...], preferred_element_type=_F32)
        z = jnp.maximum(z + bd1_ref[...], 0.0)
        o_ref[...] = jnp.dot(z, wd2_ref[...],
                             preferred_element_type=_F32) + bd2_ref[...]


def _head(h2, Wd1, bd1, Wd2, bd2):
    bm = 1000
    grid = (N // bm,)
    return pl.pallas_call(
        _head_body,
        grid=grid,
        in_specs=[pl.BlockSpec((bm, HC), lambda i: (i, 0)),
                  pl.BlockSpec((HC, 2 * HC), lambda i: (0, 0)),
                  pl.BlockSpec((1, 2 * HC), lambda i: (0, 0)),
                  pl.BlockSpec((2 * HC, 2), lambda i: (0, 0)),
                  pl.BlockSpec((1, 2), lambda i: (0, 0))],
        out_specs=pl.BlockSpec((1, 2), lambda i: (0, 0)),
        out_shape=jax.ShapeDtypeStruct((1, 2), _F32),
        scratch_shapes=[pltpu.VMEM((8, 128), _F32)],
    )(h2, Wd1, bd1, Wd2, bd2)


def kernel(x, edge_index, edge_attr, W0, att_src0, att_dst0, We0, att_e0, b0,
           W1, att_src1, att_dst1, We1, att_e1, b1, Wd1, bd1, Wd2, bd2):
    src = edge_index[0]
    dst = edge_index[1]

    pad = EPAD - E
    srcp = jnp.concatenate([src, jnp.zeros((pad,), src.dtype)])
    dstp = jnp.concatenate([dst, jnp.full((pad,), N, dst.dtype)])
    eap = jnp.concatenate([edge_attr, jnp.zeros((pad, ED), _F32)])
    z64 = jnp.zeros((RPWH, HH), _F32)
    z16 = jnp.zeros((RPW, 16), _F32)
    z8 = jnp.zeros((RPWH, 8), _F32)
    asf0 = att_src0.reshape(1, HC)
    adf0 = att_dst0.reshape(1, HC)
    aef0 = att_e0.reshape(1, HC)
    asf1 = att_src1.reshape(1, HC)
    adf1 = att_dst1.reshape(1, HC)
    aef1 = att_e1.reshape(1, HC)

    aeEp, aemax = _edge_logits(eap, We0, aef0, We1, aef1)

    seg = _sc_stats(aeEp, dstp, z16)
    seg0 = seg[0, :N, :]
    seg1 = seg[1, :N, :]

    def bound(mx8, ae4):
        bnd = mx8[:, 0:4] + mx8[:, 4:8] + jnp.maximum(ae4, 0.0)
        return jnp.where(bnd >= 0, bnd, 0.2 * bnd)

    def layer(lcol, xin, W, asf, adf, b):
        hs0, hs1, aux, mx8 = _node_matmul(xin, W, asf, adf)
        bnd = bound(mx8, aemax[:, 4 * lcol:4 * lcol + 4])
        bnd16 = jnp.pad(bnd, ((0, 0), (0, 12)))[0]
        accA, denA = _sc_layers[lcol][0](hs0, hs1, aux, srcp, dstp, aeEp,
                                         bnd16, z64, z8)
        accB, denB = _sc_layers[lcol][1](hs0, hs1, aux, srcp, dstp, aeEp,
                                         bnd16, z64, z8)
        acc = jnp.concatenate(
            [jnp.concatenate([accA[0], accA[1]], axis=1),
             jnp.concatenate([accB[0], accB[1]], axis=1)], axis=0)
        den = jnp.concatenate([denA[0] + denA[1], denB[0] + denB[1]],
                              axis=0)
        return _combine(lcol, acc, den, hs0, hs1, aux,
                        seg0, seg1, bnd, b.reshape(1, HC))

    h1 = layer(0, x, W0, asf0, adf0, b0)
    h2 = layer(1, h1, W1, asf1, adf1, b1)

    return _head(h2, Wd1, bd1.reshape(1, 2 * HC), Wd2, bd2.reshape(1, 2))
